# Initial kernel scaffold; baseline (speedup 1.0000x reference)
import jax
import jax.numpy as jnp
from jax.experimental import pallas as pl


def kernel(points, features, center_indices):
    raise NotImplementedError("write your pallas kernel here")


# speedup vs baseline: 6.4070x; 6.4070x over previous
import functools

import jax
import jax.numpy as jnp
import numpy as np
from jax import lax
from jax.experimental import pallas as pl
from jax.experimental.pallas import tpu as pltpu

_MIN_RADIUS = 0.05
_MAX_RADIUS = 0.3
_K = 16
_DENOM = 4.0 / 3.0 * np.pi * _MIN_RADIUS ** 3 + 1e-08


def _center_coords_and_dist(pts, ci, n):
    mb = ci.shape[0]
    iota_n = lax.broadcasted_iota(jnp.int32, (mb, n), 1)
    eq = iota_n == ci[:, None]
    px = pts[0:1, :]
    py = pts[1:2, :]
    pz = pts[2:3, :]
    zero = jnp.float32(0.0)
    cx = jnp.sum(jnp.where(eq, px, zero), axis=-1, keepdims=True)
    cy = jnp.sum(jnp.where(eq, py, zero), axis=-1, keepdims=True)
    cz = jnp.sum(jnp.where(eq, pz, zero), axis=-1, keepdims=True)
    dx = px - cx
    dy = py - cy
    dz = pz - cz
    d2 = (dx * dx + dy * dy) + dz * dz
    return jnp.sqrt(d2), iota_n


def _counts_body(pts_ref, ci_ref, counts_ref):
    n = pts_ref.shape[2]
    dist, _ = _center_coords_and_dist(pts_ref[0], ci_ref[0, :, 0], n)
    mask = (dist < jnp.float32(_MIN_RADIUS)).astype(jnp.float32)
    counts_ref[0, :, 0] = jnp.sum(mask, axis=-1)


def _topk_body(pts_ref, ci_ref, cnt_ref, dmax_ref, out_ref):
    n = pts_ref.shape[2]
    dist, iota_n = _center_coords_and_dist(pts_ref[0], ci_ref[0, :, 0], n)
    counts = cnt_ref[0, :, 0]
    density = counts / jnp.float32(_DENOM)
    radii = jnp.float32(_MIN_RADIUS) + jnp.float32(_MAX_RADIUS - _MIN_RADIUS) * (
        jnp.float32(1.0) - density / dmax_ref[0, 0]
    )
    d = jnp.where(dist < radii[:, None], dist, jnp.float32(1e10))
    big_idx = jnp.int32(n)
    for k in range(_K):
        m = jnp.min(d, axis=-1, keepdims=True)
        idx = jnp.min(jnp.where(d == m, iota_n, big_idx), axis=-1)
        out_ref[0, :, k] = idx
        d = jnp.where(iota_n == idx[:, None], jnp.float32(np.inf), d)


def _counts_call(pts_t, ci3, mb):
    b, _, n = pts_t.shape
    m = ci3.shape[1]
    return pl.pallas_call(
        _counts_body,
        grid=(b, m // mb),
        in_specs=[
            pl.BlockSpec((1, 3, n), lambda bi, mi: (bi, 0, 0)),
            pl.BlockSpec((1, mb, 1), lambda bi, mi: (bi, mi, 0)),
        ],
        out_specs=pl.BlockSpec((1, mb, 1), lambda bi, mi: (bi, mi, 0)),
        out_shape=jax.ShapeDtypeStruct((b, m, 1), jnp.float32),
    )(pts_t, ci3)


def _topk_call(pts_t, ci3, counts3, dmax_arr, mb):
    b, _, n = pts_t.shape
    m = ci3.shape[1]
    return pl.pallas_call(
        _topk_body,
        grid=(b, m // mb),
        in_specs=[
            pl.BlockSpec((1, 3, n), lambda bi, mi: (bi, 0, 0)),
            pl.BlockSpec((1, mb, 1), lambda bi, mi: (bi, mi, 0)),
            pl.BlockSpec((1, mb, 1), lambda bi, mi: (bi, mi, 0)),
            pl.BlockSpec(memory_space=pltpu.SMEM),
        ],
        out_specs=pl.BlockSpec((1, mb, _K), lambda bi, mi: (bi, mi, 0)),
        out_shape=jax.ShapeDtypeStruct((b, m, _K), jnp.int32),
    )(pts_t, ci3, counts3, dmax_arr)


def _sc_gather(table, gidx, chunk=512):
    from jax.experimental.pallas import tpu_sc as plsc

    info = plsc.get_sparse_core_info()
    nw = info.num_cores * info.num_subcores
    btot, c = gidx.shape[0], table.shape[1]
    b_per_w = btot // nw
    nch = b_per_w // chunk
    mesh = plsc.VectorSubcoreMesh(core_axis_name="c", subcore_axis_name="s")

    @functools.partial(
        pl.kernel,
        mesh=mesh,
        out_type=jax.ShapeDtypeStruct((btot, c), jnp.float32),
        scratch_types=[
            pltpu.VMEM((chunk,), jnp.int32),
            pltpu.VMEM((chunk, c), jnp.float32),
            pltpu.SemaphoreType.DMA,
        ],
        compiler_params=pltpu.CompilerParams(use_tc_tiling_on_sc=False),
    )
    def gather_k(table_hbm, idx_hbm, out_hbm, idx_v, rows_v, sem):
        wid = lax.axis_index("s") * info.num_cores + lax.axis_index("c")
        base = wid * b_per_w
        for j in range(nch):
            off = base + j * chunk
            pltpu.sync_copy(idx_hbm.at[pl.ds(off, chunk)], idx_v)
            pltpu.async_copy(table_hbm.at[idx_v], rows_v, sem).wait()
            pltpu.sync_copy(rows_v, out_hbm.at[pl.ds(off, chunk)])

    return gather_k(table, gidx)


def kernel(points, features, center_indices):
    b, n, _ = points.shape
    m = center_indices.shape[1]
    c = features.shape[2]

    pts_t = jnp.transpose(points, (0, 2, 1))
    ci3 = center_indices.reshape(b, m, 1)

    counts3 = _counts_call(pts_t, ci3, mb=64)
    density = counts3.reshape(b, m) / np.float32(_DENOM)
    dmax = (density.max() + np.float32(1e-8)).reshape(1, 1)

    knn = _topk_call(pts_t, ci3, counts3, dmax, mb=16)

    offs = (jnp.arange(b, dtype=jnp.int32) * n)[:, None, None]
    gidx = (knn + offs).reshape(-1)
    table = features.reshape(b * n, c)
    out = _sc_gather(table, gidx)
    return out.reshape(b, m, _K, c)


# speedup vs baseline: 10.2656x; 1.6022x over previous
import functools

import jax
import jax.numpy as jnp
import numpy as np
from jax import lax
from jax.experimental import pallas as pl
from jax.experimental.pallas import tpu as pltpu

_MIN_RADIUS = 0.05
_MAX_RADIUS = 0.3
_K = 16
_DENOM = 4.0 / 3.0 * np.pi * _MIN_RADIUS ** 3 + 1e-08


def _center_coords_and_dist(pts, ci, n):
    mb = ci.shape[0]
    iota_n = lax.broadcasted_iota(jnp.int32, (mb, n), 1)
    eq = iota_n == ci[:, None]
    px = pts[0:1, :]
    py = pts[1:2, :]
    pz = pts[2:3, :]
    zero = jnp.float32(0.0)
    cx = jnp.sum(jnp.where(eq, px, zero), axis=-1, keepdims=True)
    cy = jnp.sum(jnp.where(eq, py, zero), axis=-1, keepdims=True)
    cz = jnp.sum(jnp.where(eq, pz, zero), axis=-1, keepdims=True)
    dx = px - cx
    dy = py - cy
    dz = pz - cz
    d2 = (dx * dx + dy * dy) + dz * dz
    return jnp.sqrt(d2), iota_n


def _counts_body(pts_ref, ci_ref, counts_ref):
    n = pts_ref.shape[2]
    dist, _ = _center_coords_and_dist(pts_ref[0], ci_ref[0, :, 0], n)
    mask = (dist < jnp.float32(_MIN_RADIUS)).astype(jnp.float32)
    counts_ref[0, :, 0] = jnp.sum(mask, axis=-1)


def _topk_body(pts_ref, ci_ref, cnt_ref, dmax_ref, out_ref):
    n = pts_ref.shape[2]
    dist, iota_n = _center_coords_and_dist(pts_ref[0], ci_ref[0, :, 0], n)
    counts = cnt_ref[0, :, 0]
    density = counts / jnp.float32(_DENOM)
    radii = jnp.float32(_MIN_RADIUS) + jnp.float32(_MAX_RADIUS - _MIN_RADIUS) * (
        jnp.float32(1.0) - density / dmax_ref[0, 0]
    )
    d = jnp.where(dist < radii[:, None], dist, jnp.float32(1e10))
    big_idx = jnp.int32(n)
    for k in range(_K):
        m = jnp.min(d, axis=-1, keepdims=True)
        idx = jnp.min(jnp.where(d == m, iota_n, big_idx), axis=-1)
        out_ref[0, :, k] = idx
        d = jnp.where(iota_n == idx[:, None], jnp.float32(np.inf), d)


def _counts_call(pts_t, ci3, mb):
    b, _, n = pts_t.shape
    m = ci3.shape[1]
    return pl.pallas_call(
        _counts_body,
        grid=(b, m // mb),
        in_specs=[
            pl.BlockSpec((1, 3, n), lambda bi, mi: (bi, 0, 0)),
            pl.BlockSpec((1, mb, 1), lambda bi, mi: (bi, mi, 0)),
        ],
        out_specs=pl.BlockSpec((1, mb, 1), lambda bi, mi: (bi, mi, 0)),
        out_shape=jax.ShapeDtypeStruct((b, m, 1), jnp.float32),
    )(pts_t, ci3)


def _topk_call(pts_t, ci3, counts3, dmax_arr, mb):
    b, _, n = pts_t.shape
    m = ci3.shape[1]
    return pl.pallas_call(
        _topk_body,
        grid=(b, m // mb),
        in_specs=[
            pl.BlockSpec((1, 3, n), lambda bi, mi: (bi, 0, 0)),
            pl.BlockSpec((1, mb, 1), lambda bi, mi: (bi, mi, 0)),
            pl.BlockSpec((1, mb, 1), lambda bi, mi: (bi, mi, 0)),
            pl.BlockSpec(memory_space=pltpu.SMEM),
        ],
        out_specs=pl.BlockSpec((1, mb, _K), lambda bi, mi: (bi, mi, 0)),
        out_shape=jax.ShapeDtypeStruct((b, m, _K), jnp.int32),
    )(pts_t, ci3, counts3, dmax_arr)


def _sc_gather(table, gidx, chunk=512):
    from jax.experimental.pallas import tpu_sc as plsc

    info = plsc.get_sparse_core_info()
    nw = info.num_cores * info.num_subcores
    btot, c = gidx.shape[0], table.shape[1]
    b_per_w = btot // nw
    nch = b_per_w // chunk
    mesh = plsc.VectorSubcoreMesh(core_axis_name="c", subcore_axis_name="s")

    @functools.partial(
        pl.kernel,
        mesh=mesh,
        out_type=jax.ShapeDtypeStruct((btot, c), jnp.float32),
        scratch_types=[
            pltpu.VMEM((chunk,), jnp.int32),
            pltpu.VMEM((chunk, c), jnp.float32),
            pltpu.SemaphoreType.DMA,
        ],
        compiler_params=pltpu.CompilerParams(use_tc_tiling_on_sc=False),
    )
    def gather_k(table_hbm, idx_hbm, out_hbm, idx_v, rows_v, sem):
        wid = lax.axis_index("s") * info.num_cores + lax.axis_index("c")
        base = wid * b_per_w
        for j in range(nch):
            off = base + j * chunk
            pltpu.sync_copy(idx_hbm.at[pl.ds(off, chunk)], idx_v)
            pltpu.async_copy(table_hbm.at[idx_v], rows_v, sem).wait()
            pltpu.sync_copy(rows_v, out_hbm.at[pl.ds(off, chunk)])

    return gather_k(table, gidx)


def kernel(points, features, center_indices):
    b, n, _ = points.shape
    m = center_indices.shape[1]
    c = features.shape[2]

    pts_t = jnp.transpose(points, (0, 2, 1))
    ci3 = center_indices.reshape(b, m, 1)

    counts3 = _counts_call(pts_t, ci3, mb=128)
    density = counts3.reshape(b, m) / np.float32(_DENOM)
    dmax = (density.max() + np.float32(1e-8)).reshape(1, 1)

    knn = _topk_call(pts_t, ci3, counts3, dmax, mb=64)

    offs = (jnp.arange(b, dtype=jnp.int32) * n)[:, None, None]
    gidx = (knn + offs).reshape(-1)
    table = features.reshape(b * n, c)
    out = _sc_gather(table, gidx)
    return out.reshape(b, m, _K, c)


# speedup vs baseline: 12.5477x; 1.2223x over previous
import functools

import jax
import jax.numpy as jnp
import numpy as np
from jax import lax
from jax.experimental import pallas as pl
from jax.experimental.pallas import tpu as pltpu

_MIN_RADIUS = 0.05
_MAX_RADIUS = 0.3
_K = 16
_DENOM = 4.0 / 3.0 * np.pi * _MIN_RADIUS ** 3 + 1e-08


def _dist_from_centers(pts, cx, cy, cz):
    px = pts[0:1, :]
    py = pts[1:2, :]
    pz = pts[2:3, :]
    dx = px - cx
    dy = py - cy
    dz = pz - cz
    d2 = (dx * dx + dy * dy) + dz * dz
    return jnp.sqrt(d2)


def _counts_body(pts_ref, ci_ref, counts_ref, ctr_ref):
    n = pts_ref.shape[2]
    mb = ci_ref.shape[1]
    pts = pts_ref[0]
    ci = ci_ref[0, :, 0]
    iota_n = lax.broadcasted_iota(jnp.int32, (mb, n), 1)
    eq = iota_n == ci[:, None]
    zero = jnp.float32(0.0)
    cx = jnp.sum(jnp.where(eq, pts[0:1, :], zero), axis=-1, keepdims=True)
    cy = jnp.sum(jnp.where(eq, pts[1:2, :], zero), axis=-1, keepdims=True)
    cz = jnp.sum(jnp.where(eq, pts[2:3, :], zero), axis=-1, keepdims=True)
    dist = _dist_from_centers(pts, cx, cy, cz)
    mask = (dist < jnp.float32(_MIN_RADIUS)).astype(jnp.float32)
    counts_ref[0, :, 0:1] = jnp.sum(mask, axis=-1, keepdims=True)
    ctr_ref[0, :, 0:1] = cx
    ctr_ref[0, :, 1:2] = cy
    ctr_ref[0, :, 2:3] = cz


def _topk_body(pts_ref, ctr_ref, cnt_ref, dmax_ref, out_ref):
    n = pts_ref.shape[2]
    mb = cnt_ref.shape[1]
    cx = ctr_ref[0, :, 0:1]
    cy = ctr_ref[0, :, 1:2]
    cz = ctr_ref[0, :, 2:3]
    dist = _dist_from_centers(pts_ref[0], cx, cy, cz)
    counts = cnt_ref[0, :, 0:1]
    density = counts / jnp.float32(_DENOM)
    radii = jnp.float32(_MIN_RADIUS) + jnp.float32(_MAX_RADIUS - _MIN_RADIUS) * (
        jnp.float32(1.0) - density / dmax_ref[0, 0]
    )
    d = jnp.where(dist < radii, dist, jnp.float32(1e10))
    iota_f = lax.broadcasted_iota(jnp.int32, (mb, n), 1).astype(jnp.float32)
    big_f = jnp.float32(n)
    for k in range(_K):
        m = jnp.min(d, axis=-1, keepdims=True)
        idx_f = jnp.min(jnp.where(d == m, iota_f, big_f), axis=-1)
        out_ref[0, :, k] = idx_f.astype(jnp.int32)
        d = jnp.where(iota_f == idx_f[:, None], jnp.float32(np.inf), d)


def _counts_call(pts_t, ci3, mb):
    b, _, n = pts_t.shape
    m = ci3.shape[1]
    return pl.pallas_call(
        _counts_body,
        grid=(b, m // mb),
        in_specs=[
            pl.BlockSpec((1, 3, n), lambda bi, mi: (bi, 0, 0)),
            pl.BlockSpec((1, mb, 1), lambda bi, mi: (bi, mi, 0)),
        ],
        out_specs=[
            pl.BlockSpec((1, mb, 1), lambda bi, mi: (bi, mi, 0)),
            pl.BlockSpec((1, mb, 3), lambda bi, mi: (bi, mi, 0)),
        ],
        out_shape=[
            jax.ShapeDtypeStruct((b, m, 1), jnp.float32),
            jax.ShapeDtypeStruct((b, m, 3), jnp.float32),
        ],
    )(pts_t, ci3)


def _topk_call(pts_t, ctrs, counts3, dmax_arr, mb):
    b, _, n = pts_t.shape
    m = ctrs.shape[1]
    return pl.pallas_call(
        _topk_body,
        grid=(b, m // mb),
        in_specs=[
            pl.BlockSpec((1, 3, n), lambda bi, mi: (bi, 0, 0)),
            pl.BlockSpec((1, mb, 3), lambda bi, mi: (bi, mi, 0)),
            pl.BlockSpec((1, mb, 1), lambda bi, mi: (bi, mi, 0)),
            pl.BlockSpec(memory_space=pltpu.SMEM),
        ],
        out_specs=pl.BlockSpec((1, mb, _K), lambda bi, mi: (bi, mi, 0)),
        out_shape=jax.ShapeDtypeStruct((b, m, _K), jnp.int32),
    )(pts_t, ctrs, counts3, dmax_arr)


def _sc_gather(table, gidx, chunk=512):
    from jax.experimental.pallas import tpu_sc as plsc

    info = plsc.get_sparse_core_info()
    nw = info.num_cores * info.num_subcores
    btot, c = gidx.shape[0], table.shape[1]
    b_per_w = btot // nw
    nch = b_per_w // chunk
    mesh = plsc.VectorSubcoreMesh(core_axis_name="c", subcore_axis_name="s")

    @functools.partial(
        pl.kernel,
        mesh=mesh,
        out_type=jax.ShapeDtypeStruct((btot, c), jnp.float32),
        scratch_types=[
            pltpu.VMEM((chunk,), jnp.int32),
            pltpu.VMEM((chunk, c), jnp.float32),
            pltpu.SemaphoreType.DMA,
        ],
        compiler_params=pltpu.CompilerParams(use_tc_tiling_on_sc=False),
    )
    def gather_k(table_hbm, idx_hbm, out_hbm, idx_v, rows_v, sem):
        wid = lax.axis_index("s") * info.num_cores + lax.axis_index("c")
        base = wid * b_per_w
        for j in range(nch):
            off = base + j * chunk
            pltpu.sync_copy(idx_hbm.at[pl.ds(off, chunk)], idx_v)
            pltpu.async_copy(table_hbm.at[idx_v], rows_v, sem).wait()
            pltpu.sync_copy(rows_v, out_hbm.at[pl.ds(off, chunk)])

    return gather_k(table, gidx)


def kernel(points, features, center_indices):
    b, n, _ = points.shape
    m = center_indices.shape[1]
    c = features.shape[2]

    pts_t = jnp.transpose(points, (0, 2, 1))
    ci3 = center_indices.reshape(b, m, 1)

    counts3, ctrs = _counts_call(pts_t, ci3, mb=128)
    density = counts3.reshape(b, m) / np.float32(_DENOM)
    dmax = (density.max() + np.float32(1e-8)).reshape(1, 1)

    knn = _topk_call(pts_t, ctrs, counts3, dmax, mb=64)

    offs = (jnp.arange(b, dtype=jnp.int32) * n)[:, None, None]
    gidx = (knn + offs).reshape(-1)
    table = features.reshape(b * n, c)
    out = _sc_gather(table, gidx)
    return out.reshape(b, m, _K, c)


# speedup vs baseline: 13.1536x; 1.0483x over previous
import functools

import jax
import jax.numpy as jnp
import numpy as np
from jax import lax
from jax.experimental import pallas as pl
from jax.experimental.pallas import tpu as pltpu

_MIN_RADIUS = 0.05
_MAX_RADIUS = 0.3
_K = 16
_DENOM = 4.0 / 3.0 * np.pi * _MIN_RADIUS ** 3 + 1e-08


def _dist_from_centers(pts, cx, cy, cz):
    px = pts[0:1, :]
    py = pts[1:2, :]
    pz = pts[2:3, :]
    dx = px - cx
    dy = py - cy
    dz = pz - cz
    d2 = (dx * dx + dy * dy) + dz * dz
    return jnp.sqrt(d2)


def _counts_body(pts_ref, ci_ref, counts_ref, ctr_ref, dmax_ref):
    n = pts_ref.shape[2]
    mb = ci_ref.shape[1]
    pts = pts_ref[0]
    ci = ci_ref[0, :, 0]
    iota_n = lax.broadcasted_iota(jnp.int32, (mb, n), 1)
    eq = iota_n == ci[:, None]
    zero = jnp.float32(0.0)
    cx = jnp.sum(jnp.where(eq, pts[0:1, :], zero), axis=-1, keepdims=True)
    cy = jnp.sum(jnp.where(eq, pts[1:2, :], zero), axis=-1, keepdims=True)
    cz = jnp.sum(jnp.where(eq, pts[2:3, :], zero), axis=-1, keepdims=True)
    dist = _dist_from_centers(pts, cx, cy, cz)
    mask = (dist < jnp.float32(_MIN_RADIUS)).astype(jnp.float32)
    counts = jnp.sum(mask, axis=-1, keepdims=True)
    counts_ref[0, :, 0:1] = counts
    ctr_ref[0, :, 0:1] = cx
    ctr_ref[0, :, 1:2] = cy
    ctr_ref[0, :, 2:3] = cz
    block_dmax = jnp.max(counts / jnp.float32(_DENOM))
    first = (pl.program_id(0) == 0) & (pl.program_id(1) == 0)

    @pl.when(first)
    def _init():
        dmax_ref[0, 0] = block_dmax

    @pl.when(jnp.logical_not(first))
    def _acc():
        dmax_ref[0, 0] = jnp.maximum(dmax_ref[0, 0], block_dmax)


def _topk_body(pts_ref, ctr_ref, cnt_ref, dmax_ref, out_ref):
    n = pts_ref.shape[2]
    mb = cnt_ref.shape[1]
    cx = ctr_ref[0, :, 0:1]
    cy = ctr_ref[0, :, 1:2]
    cz = ctr_ref[0, :, 2:3]
    dist = _dist_from_centers(pts_ref[0], cx, cy, cz)
    counts = cnt_ref[0, :, 0:1]
    density = counts / jnp.float32(_DENOM)
    dmax = dmax_ref[0, 0] + jnp.float32(1e-8)
    radii = jnp.float32(_MIN_RADIUS) + jnp.float32(_MAX_RADIUS - _MIN_RADIUS) * (
        jnp.float32(1.0) - density / dmax
    )
    d = jnp.where(dist < radii, dist, jnp.float32(1e10))
    iota_f = lax.broadcasted_iota(jnp.int32, (mb, n), 1).astype(jnp.float32)
    big_f = jnp.float32(n)
    boff = (pl.program_id(0) * n).astype(jnp.float32)
    for k in range(_K):
        m = jnp.min(d, axis=-1, keepdims=True)
        idx_f = jnp.min(jnp.where(d == m, iota_f, big_f), axis=-1)
        out_ref[0, :, k] = (idx_f + boff).astype(jnp.int32)
        d = jnp.where(iota_f == idx_f[:, None], jnp.float32(np.inf), d)


def _counts_call(pts_t, ci3, mb):
    b, _, n = pts_t.shape
    m = ci3.shape[1]
    return pl.pallas_call(
        _counts_body,
        grid=(b, m // mb),
        in_specs=[
            pl.BlockSpec((1, 3, n), lambda bi, mi: (bi, 0, 0)),
            pl.BlockSpec((1, mb, 1), lambda bi, mi: (bi, mi, 0)),
        ],
        out_specs=[
            pl.BlockSpec((1, mb, 1), lambda bi, mi: (bi, mi, 0)),
            pl.BlockSpec((1, mb, 3), lambda bi, mi: (bi, mi, 0)),
            pl.BlockSpec(memory_space=pltpu.SMEM, index_map=lambda bi, mi: (0, 0)),
        ],
        out_shape=[
            jax.ShapeDtypeStruct((b, m, 1), jnp.float32),
            jax.ShapeDtypeStruct((b, m, 3), jnp.float32),
            jax.ShapeDtypeStruct((1, 1), jnp.float32),
        ],
    )(pts_t, ci3)


def _topk_call(pts_t, ctrs, counts3, dmax_arr, mb):
    b, _, n = pts_t.shape
    m = ctrs.shape[1]
    return pl.pallas_call(
        _topk_body,
        grid=(b, m // mb),
        in_specs=[
            pl.BlockSpec((1, 3, n), lambda bi, mi: (bi, 0, 0)),
            pl.BlockSpec((1, mb, 3), lambda bi, mi: (bi, mi, 0)),
            pl.BlockSpec((1, mb, 1), lambda bi, mi: (bi, mi, 0)),
            pl.BlockSpec(memory_space=pltpu.SMEM),
        ],
        out_specs=pl.BlockSpec((1, mb, _K), lambda bi, mi: (bi, mi, 0)),
        out_shape=jax.ShapeDtypeStruct((b, m, _K), jnp.int32),
    )(pts_t, ctrs, counts3, dmax_arr)


def _sc_gather(table, gidx, chunk=512):
    from jax.experimental.pallas import tpu_sc as plsc

    info = plsc.get_sparse_core_info()
    nw = info.num_cores * info.num_subcores
    btot, c = gidx.shape[0], table.shape[1]
    b_per_w = btot // nw
    nch = b_per_w // chunk
    mesh = plsc.VectorSubcoreMesh(core_axis_name="c", subcore_axis_name="s")

    @functools.partial(
        pl.kernel,
        mesh=mesh,
        out_type=jax.ShapeDtypeStruct((btot, c), jnp.float32),
        scratch_types=[
            pltpu.VMEM((chunk,), jnp.int32),
            pltpu.VMEM((chunk, c), jnp.float32),
            pltpu.SemaphoreType.DMA,
        ],
        compiler_params=pltpu.CompilerParams(use_tc_tiling_on_sc=False),
    )
    def gather_k(table_hbm, idx_hbm, out_hbm, idx_v, rows_v, sem):
        wid = lax.axis_index("s") * info.num_cores + lax.axis_index("c")
        base = wid * b_per_w
        for j in range(nch):
            off = base + j * chunk
            pltpu.sync_copy(idx_hbm.at[pl.ds(off, chunk)], idx_v)
            pltpu.async_copy(table_hbm.at[idx_v], rows_v, sem).wait()
            pltpu.sync_copy(rows_v, out_hbm.at[pl.ds(off, chunk)])

    return gather_k(table, gidx)


def kernel(points, features, center_indices):
    b, n, _ = points.shape
    m = center_indices.shape[1]
    c = features.shape[2]

    pts_t = jnp.transpose(points, (0, 2, 1))
    ci3 = center_indices.reshape(b, m, 1)

    counts3, ctrs, dmax = _counts_call(pts_t, ci3, mb=128)

    knn = _topk_call(pts_t, ctrs, counts3, dmax, mb=128)

    gidx = knn.reshape(-1)
    table = features.reshape(b * n, c)
    out = _sc_gather(table, gidx)
    return out.reshape(b, m, _K, c)
